# Initial kernel scaffold; baseline (speedup 1.0000x reference)
import jax
import jax.numpy as jnp
from jax.experimental import pallas as pl


def kernel(x, edge_index, Wl1, bl1, Wr1, br1, att1, b1, Wl2, bl2, Wr2, br2, att2, b2):
    raise NotImplementedError("write your pallas kernel here")


# speedup vs baseline: 1.1536x; 1.1536x over previous
import functools

import jax
import jax.numpy as jnp
from jax.experimental import pallas as pl

N = 10000
E = 320000
D = 128
H = 8
C = 64
NC_OUT = 40


def _mm_body(a_ref, w_ref, b_ref, o_ref):
    o_ref[...] = (
        jnp.dot(a_ref[...], w_ref[...], preferred_element_type=jnp.float32)
        + b_ref[...]
    )


def _matmul_bias(a, w, b, block_m=2000):
    m, k = a.shape
    n = w.shape[1]
    grid = (m // block_m,)
    return pl.pallas_call(
        _mm_body,
        grid=grid,
        in_specs=[
            pl.BlockSpec((block_m, k), lambda i: (i, 0)),
            pl.BlockSpec((k, n), lambda i: (0, 0)),
            pl.BlockSpec((1, n), lambda i: (0, 0)),
        ],
        out_specs=pl.BlockSpec((block_m, n), lambda i: (i, 0)),
        out_shape=jax.ShapeDtypeStruct((m, n), jnp.float32),
    )(a, w, b.reshape(1, n))


def _edge_pass(xl, xr, src, dst, att, heads, outc):
    xl3 = xl.reshape(N, heads, outc)
    xr3 = xr.reshape(N, heads, outc)
    e = xl3[src] + xr3[dst]
    e = jnp.where(e > 0, e, 0.2 * e)
    alpha = jnp.sum(e * att[None, :, :], axis=-1)
    ex = jnp.exp(alpha)
    den = jax.ops.segment_sum(ex, dst, num_segments=N)
    num = jax.ops.segment_sum(ex[:, :, None] * xl3[src], dst, num_segments=N)
    return num.reshape(N, heads * outc), den


def kernel(x, edge_index, Wl1, bl1, Wr1, br1, att1, b1, Wl2, bl2, Wr2, br2, att2, b2):
    loops = jnp.arange(N, dtype=edge_index.dtype)
    src = jnp.concatenate([edge_index[0], loops])
    dst = jnp.concatenate([edge_index[1], loops])

    xl1 = _matmul_bias(x, Wl1, bl1)
    xr1 = _matmul_bias(x, Wr1, br1)
    num1, den1 = _edge_pass(xl1, xr1, src, dst, att1, H, C)
    h = num1 / (den1[:, :, None] + 1e-16).repeat(C, axis=2).reshape(N, H * C)
    h = jax.nn.relu(h + b1)

    xl2 = _matmul_bias(h, Wl2, bl2)
    xr2 = _matmul_bias(h, Wr2, br2)
    num2, den2 = _edge_pass(xl2, xr2, src, dst, att2, 1, NC_OUT)
    out = num2 / (den2 + 1e-16) + b2
    return out


# speedup vs baseline: 2.8447x; 2.4659x over previous
import functools

import jax
import jax.numpy as jnp
from jax import lax
from jax.experimental import pallas as pl
from jax.experimental.pallas import tpu as pltpu
from jax.experimental.pallas import tpu_sc as plsc

N = 10000
E = 320000
D = 128
H = 8
C = 64
NC_OUT = 40

N2 = 10240
EP = 330752
EPW = EP // 32
K = 32
NCH = EPW // K
ZR = N2 // 16


def _mm_body(a_ref, w_ref, b_ref, o_ref):
    o_ref[...] = (
        jnp.dot(a_ref[...], w_ref[...], preferred_element_type=jnp.float32)
        + b_ref[...]
    )


def _matmul_bias(a, w, b, block_m=2048):
    m, k = a.shape
    n = w.shape[1]
    return pl.pallas_call(
        _mm_body,
        grid=(m // block_m,),
        in_specs=[
            pl.BlockSpec((block_m, k), lambda i: (i, 0)),
            pl.BlockSpec((k, n), lambda i: (0, 0)),
            pl.BlockSpec((1, n), lambda i: (0, 0)),
        ],
        out_specs=pl.BlockSpec((block_m, n), lambda i: (i, 0)),
        out_shape=jax.ShapeDtypeStruct((m, n), jnp.float32),
    )(a, w, b.reshape(1, n))


_MESH = plsc.VectorSubcoreMesh(core_axis_name="c", subcore_axis_name="s")
_I16 = lambda: lax.iota(jnp.int32, 16)


def _sc_edge2_body(xls_hbm, xrs_hbm, src_hbm, dst_hbm, att_hbm, z_hbm,
                   acc_out_hbm,
                   idx_s, idx_d, rows_l, rows_r, msg, att_v, acc, sem):
    c = lax.axis_index("c")
    s = lax.axis_index("s")
    wid = s * 2 + c
    pltpu.sync_copy(att_hbm, att_v)
    pltpu.sync_copy(z_hbm, acc.at[pl.ds(s * ZR, ZR)])
    plsc.subcore_barrier()

    def chunk(i, carry):
        base = wid * EPW + i * K
        pltpu.sync_copy(src_hbm.at[pl.ds(base, K)], idx_s)
        pltpu.sync_copy(dst_hbm.at[pl.ds(base, K)], idx_d)
        pltpu.async_copy(xls_hbm.at[idx_s], rows_l, sem).wait()
        pltpu.async_copy(xrs_hbm.at[idx_d], rows_r, sem).wait()
        for g in range(K // 16):
            eids = _I16() + g * 16

            def col_block(c4, alpha):
                att_c = att_v[pl.ds(c4 * 16, 16)]
                for j in range(16):
                    cc = jnp.full((16,), 1, jnp.int32) * (c4 * 16 + j)
                    lv = plsc.load_gather(rows_l, [eids, cc])
                    rv = plsc.load_gather(rows_r, [eids, cc])
                    ev = lv + rv
                    ev = jnp.where(ev > 0, ev, ev * 0.2)
                    alpha = alpha + att_c[j] * ev
                return alpha

            alpha = lax.fori_loop(0, 4, col_block, jnp.zeros((16,), jnp.float32))
            ex = jnp.exp(alpha)

            def msg_block(c4, carry2):
                for j in range(16):
                    cc = jnp.full((16,), 1, jnp.int32) * (c4 * 16 + j)
                    lv = plsc.load_gather(rows_l, [eids, cc])
                    plsc.store_scatter(msg, [eids, cc], ex * lv)
                return carry2

            lax.fori_loop(0, 4, msg_block, 0)
        pltpu.sync_copy(msg, acc.at[idx_d], add=True)
        return carry

    lax.fori_loop(0, NCH, chunk, 0)
    plsc.subcore_barrier()
    pltpu.sync_copy(acc.at[pl.ds(s * ZR, ZR)],
                    acc_out_hbm.at[c, pl.ds(s * ZR, ZR)])


_sc_edge2 = functools.partial(
    pl.kernel,
    _sc_edge2_body,
    out_type=jax.ShapeDtypeStruct((2, N2, 64), jnp.float32),
    mesh=_MESH,
    scratch_types=[
        pltpu.VMEM((K,), jnp.int32),
        pltpu.VMEM((K,), jnp.int32),
        pltpu.VMEM((K, 64), jnp.float32),
        pltpu.VMEM((K, 64), jnp.float32),
        pltpu.VMEM((K, 64), jnp.float32),
        pltpu.VMEM((64,), jnp.float32),
        pltpu.VMEM_SHARED((N2, 64), jnp.float32),
        pltpu.SemaphoreType.DMA,
    ],
    compiler_params=pltpu.CompilerParams(
        needs_layout_passes=False, use_tc_tiling_on_sc=False),
)()


def _sc_edge1a_body(xl_hbm, xr_hbm, src_hbm, dst_hbm, att_hbm, z_hbm,
                    ex_out_hbm, den_out_hbm,
                    idx_s, idx_d, rows_l, rows_r, exb, att_v, den_acc, sem):
    c = lax.axis_index("c")
    s = lax.axis_index("s")
    wid = s * 2 + c
    pltpu.sync_copy(att_hbm, att_v)
    pltpu.sync_copy(z_hbm, den_acc.at[pl.ds(s * ZR, ZR)])
    for e in range(K):
        exb[e, :] = jnp.zeros((16,), jnp.float32)
    plsc.subcore_barrier()

    def chunk(i, carry):
        base = wid * EPW + i * K
        pltpu.sync_copy(src_hbm.at[pl.ds(base, K)], idx_s)
        pltpu.sync_copy(dst_hbm.at[pl.ds(base, K)], idx_d)
        pltpu.async_copy(xl_hbm.at[idx_s], rows_l, sem).wait()
        pltpu.async_copy(xr_hbm.at[idx_d], rows_r, sem).wait()
        for g in range(K // 16):
            eids = _I16() + g * 16

            def head_block(h, carry2):
                def col_block(c4, alpha):
                    cbase = h * 64 + c4 * 16
                    att_c = att_v[pl.ds(cbase, 16)]
                    for j in range(16):
                        cc = jnp.full((16,), 1, jnp.int32) * (cbase + j)
                        lv = plsc.load_gather(rows_l, [eids, cc])
                        rv = plsc.load_gather(rows_r, [eids, cc])
                        ev = lv + rv
                        ev = jnp.where(ev > 0, ev, ev * 0.2)
                        alpha = alpha + att_c[j] * ev
                    return alpha

                alpha = lax.fori_loop(0, 4, col_block,
                                      jnp.zeros((16,), jnp.float32))
                ex = jnp.exp(alpha)
                hh = jnp.full((16,), 1, jnp.int32) * h
                plsc.store_scatter(exb, [eids, hh], ex)
                return carry2

            lax.fori_loop(0, H, head_block, 0)
        pltpu.sync_copy(exb, ex_out_hbm.at[pl.ds(base, K)])
        pltpu.sync_copy(exb, den_acc.at[idx_d], add=True)
        return carry

    lax.fori_loop(0, NCH, chunk, 0)
    plsc.subcore_barrier()
    pltpu.sync_copy(den_acc.at[pl.ds(s * ZR, ZR)],
                    den_out_hbm.at[c, pl.ds(s * ZR, ZR)])


_sc_edge1a = functools.partial(
    pl.kernel,
    _sc_edge1a_body,
    out_type=(jax.ShapeDtypeStruct((EP, 16), jnp.float32),
              jax.ShapeDtypeStruct((2, N2, 16), jnp.float32)),
    mesh=_MESH,
    scratch_types=[
        pltpu.VMEM((K,), jnp.int32),
        pltpu.VMEM((K,), jnp.int32),
        pltpu.VMEM((K, 512), jnp.float32),
        pltpu.VMEM((K, 512), jnp.float32),
        pltpu.VMEM((K, 16), jnp.float32),
        pltpu.VMEM((512,), jnp.float32),
        pltpu.VMEM_SHARED((N2, 16), jnp.float32),
        pltpu.SemaphoreType.DMA,
    ],
    compiler_params=pltpu.CompilerParams(
        needs_layout_passes=False, use_tc_tiling_on_sc=False),
)()


def _sc_edge1b_body(xlf_hbm, src_hbm, dst_hbm, ex_hbm, z_hbm,
                    nacc_out_hbm,
                    idx_s, idx_d, rows, exb, msg, acc, sem):
    c = lax.axis_index("c")
    s = lax.axis_index("s")
    wid = s * 2 + c

    def cc_block(cc, carry0):
        pltpu.sync_copy(z_hbm, acc.at[pl.ds(s * ZR, ZR)])
        plsc.subcore_barrier()

        def chunk(i, carry):
            base = wid * EPW + i * K
            pltpu.sync_copy(src_hbm.at[pl.ds(base, K)], idx_s)
            pltpu.sync_copy(dst_hbm.at[pl.ds(base, K)], idx_d)
            for e2 in range(K // 16):
                sl = pl.ds(e2 * 16, 16)
                idx_s[sl] = idx_s[sl] + cc * N2
            pltpu.async_copy(xlf_hbm.at[idx_s], rows, sem).wait()
            pltpu.sync_copy(ex_hbm.at[pl.ds(base, K)], exb)
            for g in range(K // 16):
                eids = _I16() + g * 16
                ex0 = plsc.load_gather(
                    exb, [eids, jnp.full((16,), 1, jnp.int32) * (2 * cc)])
                ex1 = plsc.load_gather(
                    exb, [eids, jnp.full((16,), 1, jnp.int32) * (2 * cc + 1)])

                def mk_block(exv):
                    def blk(c8, carry2):
                        for j in range(16):
                            cc2 = jnp.full((16,), 1, jnp.int32) * (c8 * 16 + j)
                            lv = plsc.load_gather(rows, [eids, cc2])
                            plsc.store_scatter(msg, [eids, cc2], exv * lv)
                        return carry2
                    return blk

                lax.fori_loop(0, 4, mk_block(ex0), 0)
                lax.fori_loop(4, 8, mk_block(ex1), 0)
            pltpu.sync_copy(msg, acc.at[idx_d], add=True)
            return carry

        lax.fori_loop(0, NCH, chunk, 0)
        plsc.subcore_barrier()
        pltpu.sync_copy(acc.at[pl.ds(s * ZR, ZR)],
                        nacc_out_hbm.at[cc, c, pl.ds(s * ZR, ZR)])
        plsc.subcore_barrier()
        return carry0

    lax.fori_loop(0, 4, cc_block, 0)


_sc_edge1b = functools.partial(
    pl.kernel,
    _sc_edge1b_body,
    out_type=jax.ShapeDtypeStruct((4, 2, N2, 128), jnp.float32),
    mesh=_MESH,
    scratch_types=[
        pltpu.VMEM((K,), jnp.int32),
        pltpu.VMEM((K,), jnp.int32),
        pltpu.VMEM((K, 128), jnp.float32),
        pltpu.VMEM((K, 16), jnp.float32),
        pltpu.VMEM((K, 128), jnp.float32),
        pltpu.VMEM_SHARED((N2, 128), jnp.float32),
        pltpu.SemaphoreType.DMA,
    ],
    compiler_params=pltpu.CompilerParams(
        needs_layout_passes=False, use_tc_tiling_on_sc=False),
)()


def _combine1_body(a_ref, b_ref, da_ref, db_ref, bias_ref, o_ref):
    num = a_ref[...] + b_ref[...]
    den = da_ref[...] + db_ref[...]
    bm = num.shape[0]
    dexp = jnp.repeat(den[:, :H], C, axis=1)
    o_ref[...] = jnp.maximum(num / (dexp + 1e-16) + bias_ref[...], 0.0)


def _combine1(na, nb, da, db, b1, block_m=2048):
    return pl.pallas_call(
        _combine1_body,
        grid=(N2 // block_m,),
        in_specs=[
            pl.BlockSpec((block_m, 512), lambda i: (i, 0)),
            pl.BlockSpec((block_m, 512), lambda i: (i, 0)),
            pl.BlockSpec((block_m, 16), lambda i: (i, 0)),
            pl.BlockSpec((block_m, 16), lambda i: (i, 0)),
            pl.BlockSpec((1, 512), lambda i: (0, 0)),
        ],
        out_specs=pl.BlockSpec((block_m, 512), lambda i: (i, 0)),
        out_shape=jax.ShapeDtypeStruct((N2, 512), jnp.float32),
    )(na, nb, da, db, b1.reshape(1, 512))


def _combine2_body(a_ref, b_ref, bias_ref, o_ref):
    ssum = a_ref[...] + b_ref[...]
    den = ssum[:, 63:64]
    o_ref[...] = ssum[:, :NC_OUT] / (den + 1e-16) + bias_ref[...]


def _combine2(acc2, b2, block_m=2048):
    return pl.pallas_call(
        _combine2_body,
        grid=(N2 // block_m,),
        in_specs=[
            pl.BlockSpec((block_m, 64), lambda i: (i, 0)),
            pl.BlockSpec((block_m, 64), lambda i: (i, 0)),
            pl.BlockSpec((1, NC_OUT), lambda i: (0, 0)),
        ],
        out_specs=pl.BlockSpec((block_m, NC_OUT), lambda i: (i, 0)),
        out_shape=jax.ShapeDtypeStruct((N2, NC_OUT), jnp.float32),
    )(acc2[0], acc2[1], b2.reshape(1, NC_OUT))


def _edge_pass_xla(xl, xr, src, dst, att, heads, outc):
    xl3 = xl.reshape(N, heads, outc)
    xr3 = xr.reshape(N, heads, outc)
    e = xl3[src] + xr3[dst]
    e = jnp.where(e > 0, e, 0.2 * e)
    alpha = jnp.sum(e * att[None, :, :], axis=-1)
    ex = jnp.exp(alpha)
    den = jax.ops.segment_sum(ex, dst, num_segments=N)
    num = jax.ops.segment_sum(ex[:, :, None] * xl3[src], dst, num_segments=N)
    return num.reshape(N, heads * outc), den


def kernel(x, edge_index, Wl1, bl1, Wr1, br1, att1, b1, Wl2, bl2, Wr2, br2, att2, b2):
    loops = jnp.arange(N, dtype=jnp.int32)
    pad_e = jnp.full((EP - E - N,), N, jnp.int32)
    src = jnp.concatenate([edge_index[0], loops, pad_e])
    dst = jnp.concatenate([edge_index[1], loops, pad_e])

    xp = jnp.pad(x, ((0, N2 - N), (0, 0)))
    xl1 = _matmul_bias(xp, Wl1, bl1)
    xr1 = _matmul_bias(xp, Wr1, br1)
    z16 = jnp.zeros((ZR, 16), jnp.float32)
    ex1, den1 = _sc_edge1a(xl1, xr1, src, dst, att1.reshape(H * C), z16)
    xlf = xl1.reshape(N2, 4, 128).transpose(1, 0, 2).reshape(4 * N2, 128)
    z128 = jnp.zeros((ZR, 128), jnp.float32)
    nacc = _sc_edge1b(xlf, src, dst, ex1, z128)
    na = nacc[:, 0].transpose(1, 0, 2).reshape(N2, 512)
    nb = nacc[:, 1].transpose(1, 0, 2).reshape(N2, 512)
    hp = _combine1(na, nb, den1[0], den1[1], b1)

    mml = _matmul_bias(hp, Wl2, bl2)
    mmr = _matmul_bias(hp, Wr2, br2)
    ones = jnp.ones((N2, 1), jnp.float32)
    zer = jnp.zeros((N2, 23), jnp.float32)
    xls2 = jnp.concatenate([mml, zer, ones], axis=1)
    xrs2 = jnp.concatenate([mmr, jnp.zeros((N2, 24), jnp.float32)], axis=1)
    attp2 = jnp.concatenate([att2.reshape(NC_OUT),
                             jnp.zeros((24,), jnp.float32)])
    z64 = jnp.zeros((ZR, 64), jnp.float32)
    acc2 = _sc_edge2(xls2, xrs2, src, dst, attp2, z64)
    out = _combine2(acc2, b2)
    return out[:N]


# speedup vs baseline: 3.6214x; 1.2730x over previous
import functools

import jax
import jax.numpy as jnp
from jax import lax
from jax.experimental import pallas as pl
from jax.experimental.pallas import tpu as pltpu
from jax.experimental.pallas import tpu_sc as plsc

N = 10000
E = 320000
D = 128
H = 8
C = 64
NC_OUT = 40

N2 = 10240
EP = 331776
EPP = EP + 128
EPW = EP // 32
KA = 32
KB = 64
ZR = N2 // 16


def _mm_body(a_ref, w_ref, b_ref, o_ref):
    o_ref[...] = (
        jnp.dot(a_ref[...], w_ref[...], preferred_element_type=jnp.float32)
        + b_ref[...]
    )


def _matmul_bias(a, w, b, block_m=2048):
    m, k = a.shape
    n = w.shape[1]
    return pl.pallas_call(
        _mm_body,
        grid=(m // block_m,),
        in_specs=[
            pl.BlockSpec((block_m, k), lambda i: (i, 0)),
            pl.BlockSpec((k, n), lambda i: (0, 0)),
            pl.BlockSpec((1, n), lambda i: (0, 0)),
        ],
        out_specs=pl.BlockSpec((block_m, n), lambda i: (i, 0)),
        out_shape=jax.ShapeDtypeStruct((m, n), jnp.float32),
    )(a, w, b.reshape(1, n))


_MESH = plsc.VectorSubcoreMesh(core_axis_name="c", subcore_axis_name="s")
_I16 = lambda: lax.iota(jnp.int32, 16)
_SC_PARAMS = pltpu.CompilerParams(
    needs_layout_passes=False, use_tc_tiling_on_sc=False)


def _mk_pipeline(src_hbm, dst_hbm, idx_s, idx_d, sem_i, wid, epw, k):
    def issue_idx(j, b):
        base = wid * epw + j * k
        pltpu.async_copy(src_hbm.at[pl.ds(base, k)], idx_s[b], sem_i[b])
        pltpu.async_copy(dst_hbm.at[pl.ds(base, k)], idx_d[b], sem_i[b])

    def wait_idx(b):
        pltpu.make_async_copy(src_hbm.at[pl.ds(0, k)], idx_s[b],
                              sem_i[b]).wait()
        pltpu.make_async_copy(dst_hbm.at[pl.ds(0, k)], idx_d[b],
                              sem_i[b]).wait()

    return issue_idx, wait_idx


def _sc_edge1a_body(xl_hbm, xr_hbm, src_hbm, dst_hbm, att_hbm, z_hbm,
                    ex_out_hbm, den_out_hbm,
                    idx_s0, idx_s1, idx_d0, idx_d1,
                    rows_l0, rows_l1, rows_r0, rows_r1,
                    exb, att_v, den_acc,
                    sem_i0, sem_i1, sem_g0, sem_g1):
    c = lax.axis_index("c")
    s = lax.axis_index("s")
    wid = s * 2 + c
    idx_s = [idx_s0, idx_s1]
    idx_d = [idx_d0, idx_d1]
    rows_l = [rows_l0, rows_l1]
    rows_r = [rows_r0, rows_r1]
    sem_i = [sem_i0, sem_i1]
    sem_g = [sem_g0, sem_g1]
    issue_idx, wait_idx = _mk_pipeline(src_hbm, dst_hbm, idx_s, idx_d,
                                       sem_i, wid, EPW, KA)

    def issue_gather(b):
        pltpu.async_copy(xl_hbm.at[idx_s[b]], rows_l[b], sem_g[b])
        pltpu.async_copy(xr_hbm.at[idx_d[b]], rows_r[b], sem_g[b])

    def wait_gather(b):
        pltpu.make_async_copy(xl_hbm.at[pl.ds(0, KA)], rows_l[b],
                              sem_g[b]).wait()
        pltpu.make_async_copy(xr_hbm.at[pl.ds(0, KA)], rows_r[b],
                              sem_g[b]).wait()

    pltpu.sync_copy(att_hbm, att_v)
    pltpu.sync_copy(z_hbm, den_acc.at[pl.ds(s * ZR, ZR)])
    for e in range(KA):
        exb[e, :] = jnp.zeros((16,), jnp.float32)
    plsc.subcore_barrier()

    issue_idx(0, 0)
    wait_idx(0)
    issue_gather(0)
    issue_idx(1, 1)

    def pair(p, carry):
        for b in range(2):
            i = 2 * p + b
            nb = 1 - b
            wait_idx(nb)
            issue_gather(nb)
            wait_gather(b)
            for g in range(KA // 16):
                eids = _I16() + g * 16

                def head_block(h, carry2):
                    def col_block(c4, alpha):
                        cbase = h * 64 + c4 * 16
                        att_c = att_v[pl.ds(cbase, 16)]
                        for j in range(16):
                            cc = jnp.full((16,), 1, jnp.int32) * (cbase + j)
                            lv = plsc.load_gather(rows_l[b], [eids, cc])
                            rv = plsc.load_gather(rows_r[b], [eids, cc])
                            ev = lv + rv
                            ev = jnp.where(ev > 0, ev, ev * 0.2)
                            alpha = alpha + att_c[j] * ev
                        return alpha

                    alpha = lax.fori_loop(0, 4, col_block,
                                          jnp.zeros((16,), jnp.float32))
                    ex = jnp.exp(alpha)
                    hh = jnp.full((16,), 1, jnp.int32) * h
                    plsc.store_scatter(exb, [eids, hh], ex)
                    return carry2

                lax.fori_loop(0, H, head_block, 0)
            base = wid * EPW + i * KA
            pltpu.sync_copy(exb, ex_out_hbm.at[pl.ds(base, KA)])
            pltpu.sync_copy(exb, den_acc.at[idx_d[b]], add=True)
            issue_idx(i + 2, b)
        return carry

    lax.fori_loop(0, (EPW // KA) // 2, pair, 0)
    wait_idx(1)
    wait_gather(0)
    plsc.subcore_barrier()
    pltpu.sync_copy(den_acc.at[pl.ds(s * ZR, ZR)],
                    den_out_hbm.at[c, pl.ds(s * ZR, ZR)])


_sc_edge1a = functools.partial(
    pl.kernel,
    _sc_edge1a_body,
    out_type=(jax.ShapeDtypeStruct((EPP, 16), jnp.float32),
              jax.ShapeDtypeStruct((2, N2, 16), jnp.float32)),
    mesh=_MESH,
    scratch_types=[
        pltpu.VMEM((KA,), jnp.int32), pltpu.VMEM((KA,), jnp.int32),
        pltpu.VMEM((KA,), jnp.int32), pltpu.VMEM((KA,), jnp.int32),
        pltpu.VMEM((KA, 512), jnp.float32), pltpu.VMEM((KA, 512), jnp.float32),
        pltpu.VMEM((KA, 512), jnp.float32), pltpu.VMEM((KA, 512), jnp.float32),
        pltpu.VMEM((KA, 16), jnp.float32),
        pltpu.VMEM((512,), jnp.float32),
        pltpu.VMEM_SHARED((N2, 16), jnp.float32),
        pltpu.SemaphoreType.DMA, pltpu.SemaphoreType.DMA,
        pltpu.SemaphoreType.DMA, pltpu.SemaphoreType.DMA,
    ],
    compiler_params=_SC_PARAMS,
)()


def _sc_edge1b_body(xlf_hbm, src_hbm, dst_hbm, ex_hbm, z_hbm,
                    nacc_out_hbm,
                    idx_s0, idx_s1, idx_d0, idx_d1,
                    rows0, rows1, exb0, exb1, msg, acc,
                    sem_i0, sem_i1, sem_g0, sem_g1):
    c = lax.axis_index("c")
    s = lax.axis_index("s")
    wid = s * 2 + c
    idx_s = [idx_s0, idx_s1]
    idx_d = [idx_d0, idx_d1]
    rows = [rows0, rows1]
    exb = [exb0, exb1]
    sem_i = [sem_i0, sem_i1]
    sem_g = [sem_g0, sem_g1]
    issue_idx, wait_idx = _mk_pipeline(src_hbm, dst_hbm, idx_s, idx_d,
                                       sem_i, wid, EPW, KB)

    def adjust_idx(b, cc):
        off = cc * N2
        for e2 in range(KB // 16):
            sl = pl.ds(e2 * 16, 16)
            idx_s[b][sl] = idx_s[b][sl] + off

    def issue_gather(j, b):
        base = wid * EPW + j * KB
        pltpu.async_copy(xlf_hbm.at[idx_s[b]], rows[b], sem_g[b])
        pltpu.async_copy(ex_hbm.at[pl.ds(base, KB)], exb[b], sem_g[b])

    def wait_gather(b):
        pltpu.make_async_copy(xlf_hbm.at[pl.ds(0, KB)], rows[b],
                              sem_g[b]).wait()
        pltpu.make_async_copy(ex_hbm.at[pl.ds(0, KB)], exb[b],
                              sem_g[b]).wait()

    def cc_block(cc, carry0):
        pltpu.sync_copy(z_hbm, acc.at[pl.ds(s * ZR, ZR)])
        plsc.subcore_barrier()

        issue_idx(0, 0)
        wait_idx(0)
        adjust_idx(0, cc)
        issue_gather(0, 0)
        issue_idx(1, 1)

        def pair(p, carry):
            for b in range(2):
                i = 2 * p + b
                nb = 1 - b
                wait_idx(nb)
                adjust_idx(nb, cc)
                issue_gather(i + 1, nb)
                wait_gather(b)
                for g in range(KB // 16):
                    eids = _I16() + g * 16
                    ex0 = plsc.load_gather(
                        exb[b],
                        [eids, jnp.full((16,), 1, jnp.int32) * (2 * cc)])
                    ex1 = plsc.load_gather(
                        exb[b],
                        [eids, jnp.full((16,), 1, jnp.int32) * (2 * cc + 1)])

                    def mk_block(exv):
                        def blk(c8, carry2):
                            for j in range(16):
                                cc2 = (jnp.full((16,), 1, jnp.int32)
                                       * (c8 * 16 + j))
                                lv = plsc.load_gather(rows[b], [eids, cc2])
                                plsc.store_scatter(msg, [eids, cc2], exv * lv)
                            return carry2
                        return blk

                    lax.fori_loop(0, 4, mk_block(ex0), 0)
                    lax.fori_loop(4, 8, mk_block(ex1), 0)
                pltpu.sync_copy(msg, acc.at[idx_d[b]], add=True)
                issue_idx(i + 2, b)
            return carry

        lax.fori_loop(0, (EPW // KB) // 2, pair, 0)
        wait_idx(1)
        wait_gather(0)
        plsc.subcore_barrier()
        pltpu.sync_copy(acc.at[pl.ds(s * ZR, ZR)],
                        nacc_out_hbm.at[cc, c, pl.ds(s * ZR, ZR)])
        plsc.subcore_barrier()
        return carry0

    lax.fori_loop(0, 4, cc_block, 0)


_sc_edge1b = functools.partial(
    pl.kernel,
    _sc_edge1b_body,
    out_type=jax.ShapeDtypeStruct((4, 2, N2, 128), jnp.float32),
    mesh=_MESH,
    scratch_types=[
        pltpu.VMEM((KB,), jnp.int32), pltpu.VMEM((KB,), jnp.int32),
        pltpu.VMEM((KB,), jnp.int32), pltpu.VMEM((KB,), jnp.int32),
        pltpu.VMEM((KB, 128), jnp.float32), pltpu.VMEM((KB, 128), jnp.float32),
        pltpu.VMEM((KB, 16), jnp.float32), pltpu.VMEM((KB, 16), jnp.float32),
        pltpu.VMEM((KB, 128), jnp.float32),
        pltpu.VMEM_SHARED((N2, 128), jnp.float32),
        pltpu.SemaphoreType.DMA, pltpu.SemaphoreType.DMA,
        pltpu.SemaphoreType.DMA, pltpu.SemaphoreType.DMA,
    ],
    compiler_params=_SC_PARAMS,
)()


def _sc_edge2_body(xls_hbm, xrs_hbm, src_hbm, dst_hbm, att_hbm, z_hbm,
                   acc_out_hbm,
                   idx_s0, idx_s1, idx_d0, idx_d1,
                   rows_l0, rows_l1, rows_r0, rows_r1,
                   msg, att_v, acc,
                   sem_i0, sem_i1, sem_g0, sem_g1):
    c = lax.axis_index("c")
    s = lax.axis_index("s")
    wid = s * 2 + c
    idx_s = [idx_s0, idx_s1]
    idx_d = [idx_d0, idx_d1]
    rows_l = [rows_l0, rows_l1]
    rows_r = [rows_r0, rows_r1]
    sem_i = [sem_i0, sem_i1]
    sem_g = [sem_g0, sem_g1]
    issue_idx, wait_idx = _mk_pipeline(src_hbm, dst_hbm, idx_s, idx_d,
                                       sem_i, wid, EPW, KB)

    def issue_gather(b):
        pltpu.async_copy(xls_hbm.at[idx_s[b]], rows_l[b], sem_g[b])
        pltpu.async_copy(xrs_hbm.at[idx_d[b]], rows_r[b], sem_g[b])

    def wait_gather(b):
        pltpu.make_async_copy(xls_hbm.at[pl.ds(0, KB)], rows_l[b],
                              sem_g[b]).wait()
        pltpu.make_async_copy(xrs_hbm.at[pl.ds(0, KB)], rows_r[b],
                              sem_g[b]).wait()

    pltpu.sync_copy(att_hbm, att_v)
    pltpu.sync_copy(z_hbm, acc.at[pl.ds(s * ZR, ZR)])
    plsc.subcore_barrier()

    issue_idx(0, 0)
    wait_idx(0)
    issue_gather(0)
    issue_idx(1, 1)

    def pair(p, carry):
        for b in range(2):
            i = 2 * p + b
            nb = 1 - b
            wait_idx(nb)
            issue_gather(nb)
            wait_gather(b)
            for g in range(KB // 16):
                eids = _I16() + g * 16

                def col_block(c4, alpha):
                    att_c = att_v[pl.ds(c4 * 16, 16)]
                    for j in range(16):
                        cc = jnp.full((16,), 1, jnp.int32) * (c4 * 16 + j)
                        lv = plsc.load_gather(rows_l[b], [eids, cc])
                        rv = plsc.load_gather(rows_r[b], [eids, cc])
                        ev = lv + rv
                        ev = jnp.where(ev > 0, ev, ev * 0.2)
                        alpha = alpha + att_c[j] * ev
                    return alpha

                alpha = lax.fori_loop(0, 4, col_block,
                                      jnp.zeros((16,), jnp.float32))
                ex = jnp.exp(alpha)

                def msg_block(c4, carry2):
                    for j in range(16):
                        cc = jnp.full((16,), 1, jnp.int32) * (c4 * 16 + j)
                        lv = plsc.load_gather(rows_l[b], [eids, cc])
                        plsc.store_scatter(msg, [eids, cc], ex * lv)
                    return carry2

                lax.fori_loop(0, 4, msg_block, 0)
            pltpu.sync_copy(msg, acc.at[idx_d[b]], add=True)
            issue_idx(i + 2, b)
        return carry

    lax.fori_loop(0, (EPW // KB) // 2, pair, 0)
    wait_idx(1)
    wait_gather(0)
    plsc.subcore_barrier()
    pltpu.sync_copy(acc.at[pl.ds(s * ZR, ZR)],
                    acc_out_hbm.at[c, pl.ds(s * ZR, ZR)])


_sc_edge2 = functools.partial(
    pl.kernel,
    _sc_edge2_body,
    out_type=jax.ShapeDtypeStruct((2, N2, 64), jnp.float32),
    mesh=_MESH,
    scratch_types=[
        pltpu.VMEM((KB,), jnp.int32), pltpu.VMEM((KB,), jnp.int32),
        pltpu.VMEM((KB,), jnp.int32), pltpu.VMEM((KB,), jnp.int32),
        pltpu.VMEM((KB, 64), jnp.float32), pltpu.VMEM((KB, 64), jnp.float32),
        pltpu.VMEM((KB, 64), jnp.float32), pltpu.VMEM((KB, 64), jnp.float32),
        pltpu.VMEM((KB, 64), jnp.float32),
        pltpu.VMEM((64,), jnp.float32),
        pltpu.VMEM_SHARED((N2, 64), jnp.float32),
        pltpu.SemaphoreType.DMA, pltpu.SemaphoreType.DMA,
        pltpu.SemaphoreType.DMA, pltpu.SemaphoreType.DMA,
    ],
    compiler_params=_SC_PARAMS,
)()


def _combine1_body(a_ref, b_ref, da_ref, db_ref, bias_ref, o_ref):
    num = a_ref[...] + b_ref[...]
    den = da_ref[...] + db_ref[...]
    dexp = jnp.repeat(den[:, :H], C, axis=1)
    o_ref[...] = jnp.maximum(num / (dexp + 1e-16) + bias_ref[...], 0.0)


def _combine1(na, nb, da, db, b1, block_m=2048):
    return pl.pallas_call(
        _combine1_body,
        grid=(N2 // block_m,),
        in_specs=[
            pl.BlockSpec((block_m, 512), lambda i: (i, 0)),
            pl.BlockSpec((block_m, 512), lambda i: (i, 0)),
            pl.BlockSpec((block_m, 16), lambda i: (i, 0)),
            pl.BlockSpec((block_m, 16), lambda i: (i, 0)),
            pl.BlockSpec((1, 512), lambda i: (0, 0)),
        ],
        out_specs=pl.BlockSpec((block_m, 512), lambda i: (i, 0)),
        out_shape=jax.ShapeDtypeStruct((N2, 512), jnp.float32),
    )(na, nb, da, db, b1.reshape(1, 512))


def _combine2_body(a_ref, b_ref, bias_ref, o_ref):
    ssum = a_ref[...] + b_ref[...]
    den = ssum[:, 63:64]
    o_ref[...] = ssum[:, :NC_OUT] / (den + 1e-16) + bias_ref[...]


def _combine2(acc2, b2, block_m=2048):
    return pl.pallas_call(
        _combine2_body,
        grid=(N2 // block_m,),
        in_specs=[
            pl.BlockSpec((block_m, 64), lambda i: (i, 0)),
            pl.BlockSpec((block_m, 64), lambda i: (i, 0)),
            pl.BlockSpec((1, NC_OUT), lambda i: (0, 0)),
        ],
        out_specs=pl.BlockSpec((block_m, NC_OUT), lambda i: (i, 0)),
        out_shape=jax.ShapeDtypeStruct((N2, NC_OUT), jnp.float32),
    )(acc2[0], acc2[1], b2.reshape(1, NC_OUT))


def kernel(x, edge_index, Wl1, bl1, Wr1, br1, att1, b1, Wl2, bl2, Wr2, br2, att2, b2):
    loops = jnp.arange(N, dtype=jnp.int32)
    pad_e = jnp.full((EPP - E - N,), N, jnp.int32)
    src = jnp.concatenate([edge_index[0], loops, pad_e])
    dst = jnp.concatenate([edge_index[1], loops, pad_e])

    xp = jnp.pad(x, ((0, N2 - N), (0, 0)))
    xl1 = _matmul_bias(xp, Wl1, bl1)
    xr1 = _matmul_bias(xp, Wr1, br1)
    z16 = jnp.zeros((ZR, 16), jnp.float32)
    ex1, den1 = _sc_edge1a(xl1, xr1, src, dst, att1.reshape(H * C), z16)
    xlf = xl1.reshape(N2, 4, 128).transpose(1, 0, 2).reshape(4 * N2, 128)
    z128 = jnp.zeros((ZR, 128), jnp.float32)
    nacc = _sc_edge1b(xlf, src, dst, ex1, z128)
    na = nacc[:, 0].transpose(1, 0, 2).reshape(N2, 512)
    nb = nacc[:, 1].transpose(1, 0, 2).reshape(N2, 512)
    hp = _combine1(na, nb, den1[0], den1[1], b1)

    mml = _matmul_bias(hp, Wl2, bl2)
    mmr = _matmul_bias(hp, Wr2, br2)
    ones = jnp.ones((N2, 1), jnp.float32)
    zer = jnp.zeros((N2, 23), jnp.float32)
    xls2 = jnp.concatenate([mml, zer, ones], axis=1)
    xrs2 = jnp.concatenate([mmr, jnp.zeros((N2, 24), jnp.float32)], axis=1)
    attp2 = jnp.concatenate([att2.reshape(NC_OUT),
                             jnp.zeros((24,), jnp.float32)])
    z64 = jnp.zeros((ZR, 64), jnp.float32)
    acc2 = _sc_edge2(xls2, xrs2, src, dst, attp2, z64)
    out = _combine2(acc2, b2)
    return out[:N]


# speedup vs baseline: 3.6529x; 1.0087x over previous
import functools

import jax
import jax.numpy as jnp
from jax import lax
from jax.experimental import pallas as pl
from jax.experimental.pallas import tpu as pltpu
from jax.experimental.pallas import tpu_sc as plsc

N = 10000
E = 320000
D = 128
H = 8
C = 64
NC_OUT = 40

N2 = 10240
EP = 331776
EPP = EP + 256
EPW = EP // 32
KA = 32
KB = 96
ZR = N2 // 16


def _mm_body(a_ref, w_ref, b_ref, o_ref):
    o_ref[...] = (
        jnp.dot(a_ref[...], w_ref[...], preferred_element_type=jnp.float32)
        + b_ref[...]
    )


def _matmul_bias(a, w, b, block_m=2048):
    m, k = a.shape
    n = w.shape[1]
    return pl.pallas_call(
        _mm_body,
        grid=(m // block_m,),
        in_specs=[
            pl.BlockSpec((block_m, k), lambda i: (i, 0)),
            pl.BlockSpec((k, n), lambda i: (0, 0)),
            pl.BlockSpec((1, n), lambda i: (0, 0)),
        ],
        out_specs=pl.BlockSpec((block_m, n), lambda i: (i, 0)),
        out_shape=jax.ShapeDtypeStruct((m, n), jnp.float32),
    )(a, w, b.reshape(1, n))


_MESH = plsc.VectorSubcoreMesh(core_axis_name="c", subcore_axis_name="s")
_I16 = lambda: lax.iota(jnp.int32, 16)
_SC_PARAMS = pltpu.CompilerParams(
    needs_layout_passes=False, use_tc_tiling_on_sc=False)


def _mk_pipeline(src_hbm, dst_hbm, idx_s, idx_d, sem_i, wid, epw, k):
    def issue_idx(j, b):
        base = wid * epw + j * k
        pltpu.async_copy(src_hbm.at[pl.ds(base, k)], idx_s[b], sem_i[b])
        pltpu.async_copy(dst_hbm.at[pl.ds(base, k)], idx_d[b], sem_i[b])

    def wait_idx(b):
        pltpu.make_async_copy(src_hbm.at[pl.ds(0, k)], idx_s[b],
                              sem_i[b]).wait()
        pltpu.make_async_copy(dst_hbm.at[pl.ds(0, k)], idx_d[b],
                              sem_i[b]).wait()

    return issue_idx, wait_idx


def _sc_edge1a_body(xl_hbm, xr_hbm, src_hbm, dst_hbm, att_hbm, z_hbm,
                    ex_out_hbm, den_out_hbm,
                    idx_s0, idx_s1, idx_d0, idx_d1,
                    rows_l0, rows_l1, rows_r0, rows_r1,
                    exb, att_v, den_acc,
                    sem_i0, sem_i1, sem_g0, sem_g1):
    c = lax.axis_index("c")
    s = lax.axis_index("s")
    wid = s * 2 + c
    idx_s = [idx_s0, idx_s1]
    idx_d = [idx_d0, idx_d1]
    rows_l = [rows_l0, rows_l1]
    rows_r = [rows_r0, rows_r1]
    sem_i = [sem_i0, sem_i1]
    sem_g = [sem_g0, sem_g1]
    issue_idx, wait_idx = _mk_pipeline(src_hbm, dst_hbm, idx_s, idx_d,
                                       sem_i, wid, EPW, KA)

    def issue_gather(b):
        pltpu.async_copy(xl_hbm.at[idx_s[b]], rows_l[b], sem_g[b])
        pltpu.async_copy(xr_hbm.at[idx_d[b]], rows_r[b], sem_g[b])

    def wait_gather(b):
        pltpu.make_async_copy(xl_hbm.at[pl.ds(0, KA)], rows_l[b],
                              sem_g[b]).wait()
        pltpu.make_async_copy(xr_hbm.at[pl.ds(0, KA)], rows_r[b],
                              sem_g[b]).wait()

    pltpu.sync_copy(att_hbm, att_v)
    pltpu.sync_copy(z_hbm, den_acc.at[pl.ds(s * ZR, ZR)])
    for e in range(KA):
        exb[e, :] = jnp.zeros((16,), jnp.float32)
    plsc.subcore_barrier()

    issue_idx(0, 0)
    wait_idx(0)
    issue_gather(0)
    issue_idx(1, 1)

    def pair(p, carry):
        for b in range(2):
            i = 2 * p + b
            nb = 1 - b
            wait_idx(nb)
            issue_gather(nb)
            wait_gather(b)
            for g in range(KA // 16):
                eids = _I16() + g * 16

                def head_block(h, carry2):
                    def col_block(c4, alpha):
                        cbase = h * 64 + c4 * 16
                        att_c = att_v[pl.ds(cbase, 16)]
                        for j in range(16):
                            cc = jnp.full((16,), 1, jnp.int32) * (cbase + j)
                            lv = plsc.load_gather(rows_l[b], [eids, cc])
                            rv = plsc.load_gather(rows_r[b], [eids, cc])
                            ev = lv + rv
                            ev = jnp.where(ev > 0, ev, ev * 0.2)
                            alpha = alpha + att_c[j] * ev
                        return alpha

                    alpha = lax.fori_loop(0, 4, col_block,
                                          jnp.zeros((16,), jnp.float32))
                    ex = jnp.exp(alpha)
                    hh = jnp.full((16,), 1, jnp.int32) * h
                    plsc.store_scatter(exb, [eids, hh], ex)
                    return carry2

                lax.fori_loop(0, H, head_block, 0)
            base = wid * EPW + i * KA
            pltpu.sync_copy(exb, ex_out_hbm.at[pl.ds(base, KA)])
            pltpu.sync_copy(exb, den_acc.at[idx_d[b]], add=True)
            issue_idx(i + 2, b)
        return carry

    lax.fori_loop(0, (EPW // KA) // 2, pair, 0)
    wait_idx(1)
    wait_gather(0)
    plsc.subcore_barrier()
    pltpu.sync_copy(den_acc.at[pl.ds(s * ZR, ZR)],
                    den_out_hbm.at[c, pl.ds(s * ZR, ZR)])


_sc_edge1a = functools.partial(
    pl.kernel,
    _sc_edge1a_body,
    out_type=(jax.ShapeDtypeStruct((EPP, 16), jnp.float32),
              jax.ShapeDtypeStruct((2, N2, 16), jnp.float32)),
    mesh=_MESH,
    scratch_types=[
        pltpu.VMEM((KA,), jnp.int32), pltpu.VMEM((KA,), jnp.int32),
        pltpu.VMEM((KA,), jnp.int32), pltpu.VMEM((KA,), jnp.int32),
        pltpu.VMEM((KA, 512), jnp.float32), pltpu.VMEM((KA, 512), jnp.float32),
        pltpu.VMEM((KA, 512), jnp.float32), pltpu.VMEM((KA, 512), jnp.float32),
        pltpu.VMEM((KA, 16), jnp.float32),
        pltpu.VMEM((512,), jnp.float32),
        pltpu.VMEM_SHARED((N2, 16), jnp.float32),
        pltpu.SemaphoreType.DMA, pltpu.SemaphoreType.DMA,
        pltpu.SemaphoreType.DMA, pltpu.SemaphoreType.DMA,
    ],
    compiler_params=_SC_PARAMS,
)()


def _sc_edge1b_body(xlf_hbm, src_hbm, dst_hbm, ex_hbm, z_hbm,
                    nacc_out_hbm,
                    idx_s0, idx_s1, idx_d0, idx_d1,
                    rows0, rows1, exb0, exb1, msg, acc,
                    sem_i0, sem_i1, sem_g0, sem_g1):
    c = lax.axis_index("c")
    s = lax.axis_index("s")
    wid = s * 2 + c
    idx_s = [idx_s0, idx_s1]
    idx_d = [idx_d0, idx_d1]
    rows = [rows0, rows1]
    exb = [exb0, exb1]
    sem_i = [sem_i0, sem_i1]
    sem_g = [sem_g0, sem_g1]
    issue_idx, wait_idx = _mk_pipeline(src_hbm, dst_hbm, idx_s, idx_d,
                                       sem_i, wid, EPW, KB)

    def adjust_idx(b, cc):
        off = cc * N2
        for e2 in range(KB // 16):
            sl = pl.ds(e2 * 16, 16)
            idx_s[b][sl] = idx_s[b][sl] + off

    def issue_gather(j, b):
        base = wid * EPW + j * KB
        pltpu.async_copy(xlf_hbm.at[idx_s[b]], rows[b], sem_g[b])
        pltpu.async_copy(ex_hbm.at[pl.ds(base, KB)], exb[b], sem_g[b])

    def wait_gather(b):
        pltpu.make_async_copy(xlf_hbm.at[pl.ds(0, KB)], rows[b],
                              sem_g[b]).wait()
        pltpu.make_async_copy(ex_hbm.at[pl.ds(0, KB)], exb[b],
                              sem_g[b]).wait()

    def cc_block(cc, carry0):
        pltpu.sync_copy(z_hbm, acc.at[pl.ds(s * ZR, ZR)])
        plsc.subcore_barrier()

        issue_idx(0, 0)
        wait_idx(0)
        adjust_idx(0, cc)
        issue_gather(0, 0)
        issue_idx(1, 1)

        def pair(p, carry):
            for b in range(2):
                i = 2 * p + b
                nb = 1 - b
                wait_idx(nb)
                adjust_idx(nb, cc)
                issue_gather(i + 1, nb)
                wait_gather(b)
                for g in range(KB // 16):
                    eids = _I16() + g * 16
                    ex0 = plsc.load_gather(
                        exb[b],
                        [eids, jnp.full((16,), 1, jnp.int32) * (2 * cc)])
                    ex1 = plsc.load_gather(
                        exb[b],
                        [eids, jnp.full((16,), 1, jnp.int32) * (2 * cc + 1)])

                    def mk_block(exv):
                        def blk(c8, carry2):
                            for j in range(16):
                                cc2 = (jnp.full((16,), 1, jnp.int32)
                                       * (c8 * 16 + j))
                                lv = plsc.load_gather(rows[b], [eids, cc2])
                                plsc.store_scatter(msg, [eids, cc2], exv * lv)
                            return carry2
                        return blk

                    lax.fori_loop(0, 4, mk_block(ex0), 0)
                    lax.fori_loop(4, 8, mk_block(ex1), 0)
                pltpu.sync_copy(msg, acc.at[idx_d[b]], add=True)
                issue_idx(i + 2, b)
            return carry

        lax.fori_loop(0, (EPW // KB) // 2, pair, 0)
        wait_idx(1)
        wait_gather(0)
        plsc.subcore_barrier()
        pltpu.sync_copy(acc.at[pl.ds(s * ZR, ZR)],
                        nacc_out_hbm.at[cc, c, pl.ds(s * ZR, ZR)])
        plsc.subcore_barrier()
        return carry0

    lax.fori_loop(0, 4, cc_block, 0)


_sc_edge1b = functools.partial(
    pl.kernel,
    _sc_edge1b_body,
    out_type=jax.ShapeDtypeStruct((4, 2, N2, 128), jnp.float32),
    mesh=_MESH,
    scratch_types=[
        pltpu.VMEM((KB,), jnp.int32), pltpu.VMEM((KB,), jnp.int32),
        pltpu.VMEM((KB,), jnp.int32), pltpu.VMEM((KB,), jnp.int32),
        pltpu.VMEM((KB, 128), jnp.float32), pltpu.VMEM((KB, 128), jnp.float32),
        pltpu.VMEM((KB, 16), jnp.float32), pltpu.VMEM((KB, 16), jnp.float32),
        pltpu.VMEM((KB, 128), jnp.float32),
        pltpu.VMEM_SHARED((N2, 128), jnp.float32),
        pltpu.SemaphoreType.DMA, pltpu.SemaphoreType.DMA,
        pltpu.SemaphoreType.DMA, pltpu.SemaphoreType.DMA,
    ],
    compiler_params=_SC_PARAMS,
)()


def _sc_edge2_body(xls_hbm, xrs_hbm, src_hbm, dst_hbm, att_hbm, z_hbm,
                   acc_out_hbm,
                   idx_s0, idx_s1, idx_d0, idx_d1,
                   rows_l0, rows_l1, rows_r0, rows_r1,
                   msg, att_v, acc,
                   sem_i0, sem_i1, sem_g0, sem_g1):
    c = lax.axis_index("c")
    s = lax.axis_index("s")
    wid = s * 2 + c
    idx_s = [idx_s0, idx_s1]
    idx_d = [idx_d0, idx_d1]
    rows_l = [rows_l0, rows_l1]
    rows_r = [rows_r0, rows_r1]
    sem_i = [sem_i0, sem_i1]
    sem_g = [sem_g0, sem_g1]
    issue_idx, wait_idx = _mk_pipeline(src_hbm, dst_hbm, idx_s, idx_d,
                                       sem_i, wid, EPW, KB)

    def issue_gather(b):
        pltpu.async_copy(xls_hbm.at[idx_s[b]], rows_l[b], sem_g[b])
        pltpu.async_copy(xrs_hbm.at[idx_d[b]], rows_r[b], sem_g[b])

    def wait_gather(b):
        pltpu.make_async_copy(xls_hbm.at[pl.ds(0, KB)], rows_l[b],
                              sem_g[b]).wait()
        pltpu.make_async_copy(xrs_hbm.at[pl.ds(0, KB)], rows_r[b],
                              sem_g[b]).wait()

    pltpu.sync_copy(att_hbm, att_v)
    pltpu.sync_copy(z_hbm, acc.at[pl.ds(s * ZR, ZR)])
    plsc.subcore_barrier()

    issue_idx(0, 0)
    wait_idx(0)
    issue_gather(0)
    issue_idx(1, 1)

    def pair(p, carry):
        for b in range(2):
            i = 2 * p + b
            nb = 1 - b
            wait_idx(nb)
            issue_gather(nb)
            wait_gather(b)
            for g in range(KB // 16):
                eids = _I16() + g * 16

                def col_block(c4, alpha):
                    att_c = att_v[pl.ds(c4 * 16, 16)]
                    for j in range(16):
                        cc = jnp.full((16,), 1, jnp.int32) * (c4 * 16 + j)
                        lv = plsc.load_gather(rows_l[b], [eids, cc])
                        rv = plsc.load_gather(rows_r[b], [eids, cc])
                        ev = lv + rv
                        ev = jnp.where(ev > 0, ev, ev * 0.2)
                        alpha = alpha + att_c[j] * ev
                    return alpha

                alpha = lax.fori_loop(0, 4, col_block,
                                      jnp.zeros((16,), jnp.float32))
                ex = jnp.exp(alpha)

                def msg_block(c4, carry2):
                    for j in range(16):
                        cc = jnp.full((16,), 1, jnp.int32) * (c4 * 16 + j)
                        lv = plsc.load_gather(rows_l[b], [eids, cc])
                        plsc.store_scatter(msg, [eids, cc], ex * lv)
                    return carry2

                lax.fori_loop(0, 4, msg_block, 0)
            pltpu.sync_copy(msg, acc.at[idx_d[b]], add=True)
            issue_idx(i + 2, b)
        return carry

    lax.fori_loop(0, (EPW // KB) // 2, pair, 0)
    wait_idx(1)
    wait_gather(0)
    plsc.subcore_barrier()
    pltpu.sync_copy(acc.at[pl.ds(s * ZR, ZR)],
                    acc_out_hbm.at[c, pl.ds(s * ZR, ZR)])


_sc_edge2 = functools.partial(
    pl.kernel,
    _sc_edge2_body,
    out_type=jax.ShapeDtypeStruct((2, N2, 64), jnp.float32),
    mesh=_MESH,
    scratch_types=[
        pltpu.VMEM((KB,), jnp.int32), pltpu.VMEM((KB,), jnp.int32),
        pltpu.VMEM((KB,), jnp.int32), pltpu.VMEM((KB,), jnp.int32),
        pltpu.VMEM((KB, 64), jnp.float32), pltpu.VMEM((KB, 64), jnp.float32),
        pltpu.VMEM((KB, 64), jnp.float32), pltpu.VMEM((KB, 64), jnp.float32),
        pltpu.VMEM((KB, 64), jnp.float32),
        pltpu.VMEM((64,), jnp.float32),
        pltpu.VMEM_SHARED((N2, 64), jnp.float32),
        pltpu.SemaphoreType.DMA, pltpu.SemaphoreType.DMA,
        pltpu.SemaphoreType.DMA, pltpu.SemaphoreType.DMA,
    ],
    compiler_params=_SC_PARAMS,
)()


def _combine1_body(a_ref, b_ref, da_ref, db_ref, bias_ref, o_ref):
    num = a_ref[...] + b_ref[...]
    den = da_ref[...] + db_ref[...]
    dexp = jnp.repeat(den[:, :H], C, axis=1)
    o_ref[...] = jnp.maximum(num / (dexp + 1e-16) + bias_ref[...], 0.0)


def _combine1(na, nb, da, db, b1, block_m=2048):
    return pl.pallas_call(
        _combine1_body,
        grid=(N2 // block_m,),
        in_specs=[
            pl.BlockSpec((block_m, 512), lambda i: (i, 0)),
            pl.BlockSpec((block_m, 512), lambda i: (i, 0)),
            pl.BlockSpec((block_m, 16), lambda i: (i, 0)),
            pl.BlockSpec((block_m, 16), lambda i: (i, 0)),
            pl.BlockSpec((1, 512), lambda i: (0, 0)),
        ],
        out_specs=pl.BlockSpec((block_m, 512), lambda i: (i, 0)),
        out_shape=jax.ShapeDtypeStruct((N2, 512), jnp.float32),
    )(na, nb, da, db, b1.reshape(1, 512))


def _combine2_body(a_ref, b_ref, bias_ref, o_ref):
    ssum = a_ref[...] + b_ref[...]
    den = ssum[:, 63:64]
    o_ref[...] = ssum[:, :NC_OUT] / (den + 1e-16) + bias_ref[...]


def _combine2(acc2, b2, block_m=2048):
    return pl.pallas_call(
        _combine2_body,
        grid=(N2 // block_m,),
        in_specs=[
            pl.BlockSpec((block_m, 64), lambda i: (i, 0)),
            pl.BlockSpec((block_m, 64), lambda i: (i, 0)),
            pl.BlockSpec((1, NC_OUT), lambda i: (0, 0)),
        ],
        out_specs=pl.BlockSpec((block_m, NC_OUT), lambda i: (i, 0)),
        out_shape=jax.ShapeDtypeStruct((N2, NC_OUT), jnp.float32),
    )(acc2[0], acc2[1], b2.reshape(1, NC_OUT))


def kernel(x, edge_index, Wl1, bl1, Wr1, br1, att1, b1, Wl2, bl2, Wr2, br2, att2, b2):
    loops = jnp.arange(N, dtype=jnp.int32)
    pad_e = jnp.full((EPP - E - N,), N, jnp.int32)
    src = jnp.concatenate([edge_index[0], loops, pad_e])
    dst = jnp.concatenate([edge_index[1], loops, pad_e])

    xp = jnp.pad(x, ((0, N2 - N), (0, 0)))
    xl1 = _matmul_bias(xp, Wl1, bl1)
    xr1 = _matmul_bias(xp, Wr1, br1)
    z16 = jnp.zeros((ZR, 16), jnp.float32)
    ex1, den1 = _sc_edge1a(xl1, xr1, src, dst, att1.reshape(H * C), z16)
    xlf = xl1.reshape(N2, 4, 128).transpose(1, 0, 2).reshape(4 * N2, 128)
    z128 = jnp.zeros((ZR, 128), jnp.float32)
    nacc = _sc_edge1b(xlf, src, dst, ex1, z128)
    na = nacc[:, 0].transpose(1, 0, 2).reshape(N2, 512)
    nb = nacc[:, 1].transpose(1, 0, 2).reshape(N2, 512)
    hp = _combine1(na, nb, den1[0], den1[1], b1)

    mml = _matmul_bias(hp, Wl2, bl2)
    mmr = _matmul_bias(hp, Wr2, br2)
    ones = jnp.ones((N2, 1), jnp.float32)
    zer = jnp.zeros((N2, 23), jnp.float32)
    xls2 = jnp.concatenate([mml, zer, ones], axis=1)
    xrs2 = jnp.concatenate([mmr, jnp.zeros((N2, 24), jnp.float32)], axis=1)
    attp2 = jnp.concatenate([att2.reshape(NC_OUT),
                             jnp.zeros((24,), jnp.float32)])
    z64 = jnp.zeros((ZR, 64), jnp.float32)
    acc2 = _sc_edge2(xls2, xrs2, src, dst, attp2, z64)
    out = _combine2(acc2, b2)
    return out[:N]
